# Initial kernel scaffold; baseline (speedup 1.0000x reference)
import functools

import jax
import jax.numpy as jnp
from jax import lax
from jax.experimental import pallas as pl
from jax.experimental.pallas import tpu as pltpu
from jax.experimental.pallas import tpu_sc as plsc

NC = 2
NS = 16
NW = NC * NS
L = 16
D = 128


_MLP_BLK = 512


@functools.lru_cache(maxsize=None)
def _mlp_call(n_out):
    grid = (pl.cdiv(n_out, _MLP_BLK),)

    def body(a_ref, b_ref, w0_ref, w1_ref, bias_ref, o_ref):
        acc = jnp.dot(a_ref[...], w0_ref[...], preferred_element_type=jnp.float32)
        acc = acc + jnp.dot(b_ref[...], w1_ref[...], preferred_element_type=jnp.float32)
        acc = acc + bias_ref[0:1, :]
        o_ref[...] = acc * jax.nn.sigmoid(acc)

    return pl.pallas_call(
        body,
        grid=grid,
        in_specs=[
            pl.BlockSpec((_MLP_BLK, D), lambda i: (i, 0)),
            pl.BlockSpec((_MLP_BLK, D), lambda i: (i, 0)),
            pl.BlockSpec((D, D), lambda i: (0, 0)),
            pl.BlockSpec((D, D), lambda i: (0, 0)),
            pl.BlockSpec((8, D), lambda i: (0, 0)),
        ],
        out_specs=pl.BlockSpec((_MLP_BLK, D), lambda i: (i, 0)),
        out_shape=jax.ShapeDtypeStruct((n_out, D), jnp.float32),
    )


def _mlp(a, b, W, bias, n_out):
    w0 = W[:D]
    w1 = W[D:]
    bias8 = jnp.broadcast_to(bias[None, :], (8, D))
    return _mlp_call(n_out)(a, b, w0, w1, bias8)


_GB = 256


@functools.lru_cache(maxsize=None)
def _gather_call(n_pad, n_h):
    rows_per_w = n_pad // NW
    n_chunks = rows_per_w // _GB
    mesh = plsc.VectorSubcoreMesh(
        core_axis_name="c", subcore_axis_name="s", num_cores=NC, num_subcores=NS)

    @functools.partial(
        pl.kernel,
        out_type=[jax.ShapeDtypeStruct((n_pad, D), jnp.float32),
                  jax.ShapeDtypeStruct((n_pad, D), jnp.float32)],
        mesh=mesh,
        scratch_types=[
            pltpu.VMEM((_GB,), jnp.int32),
            pltpu.VMEM((_GB, D), jnp.float32),
            pltpu.SemaphoreType.DMA,
        ],
    )
    def k(h_hbm, i0_hbm, i1_hbm, g0_hbm, g1_hbm, ibuf, rbuf, sem):
        w = lax.axis_index("s") * NC + lax.axis_index("c")

        def chunk(ci, _):
            base = w * rows_per_w + ci * _GB
            for ih, gh in ((i0_hbm, g0_hbm), (i1_hbm, g1_hbm)):
                pltpu.sync_copy(ih.at[pl.ds(base, _GB)], ibuf)
                cps = [
                    pltpu.async_copy(
                        h_hbm.at[ibuf.at[pl.ds(j * 128, 128)]],
                        rbuf.at[pl.ds(j * 128, 128)], sem)
                    for j in range(_GB // 128)
                ]
                for cp in cps:
                    cp.wait()
                pltpu.sync_copy(rbuf, gh.at[pl.ds(base, _GB)])
            return 0

        lax.fori_loop(0, n_chunks, chunk, 0)

    return k


def _gather2(h, idx0, idx1):
    n = idx0.shape[0]
    n_pad = ((n + NW * _GB - 1) // (NW * _GB)) * (NW * _GB)
    pad = jnp.zeros((n_pad - n,), jnp.int32)
    i0 = jnp.concatenate([idx0.astype(jnp.int32), pad])
    i1 = jnp.concatenate([idx1.astype(jnp.int32), pad])
    return _gather_call(n_pad, h.shape[0])(h, i0, i1)


_EB = 1280
_CR = 14336


@functools.lru_cache(maxsize=None)
def _scatter_call(n_pad, n_h, m):
    pch = pl.cdiv(pl.cdiv(m, NC), _CR)
    m_pad = NC * pch * _CR
    bpt = n_pad // (_EB * NS)
    cap = 2 * bpt * _EB + 128
    mesh = plsc.VectorSubcoreMesh(
        core_axis_name="c", subcore_axis_name="s", num_cores=NC, num_subcores=NS)
    zr = _CR // NS

    @functools.partial(
        pl.kernel,
        out_type=jax.ShapeDtypeStruct((m_pad, D), jnp.float32),
        mesh=mesh,
        scratch_types=[
            pltpu.VMEM((cap,), jnp.int32),
            pltpu.VMEM((cap,), jnp.int32),
            pltpu.VMEM((_EB,), jnp.int32),
            pltpu.VMEM((128, D), jnp.float32),
            pltpu.VMEM((128, D), jnp.float32),
            pltpu.VMEM((1, 128), jnp.int32),
            pltpu.SemaphoreType.DMA,
            pltpu.VMEM_SHARED((_CR + 8, D), jnp.float32),
        ],
    )
    def k(h_hbm, c0_hbm, c1_hbm, hd_hbm, pos, off, ebuf, rbuf, zbuf, irow,
          sem, acc):
        s = lax.axis_index("c")
        t = lax.axis_index("s")
        zvec = jnp.zeros((L,), jnp.float32)

        def zb(i, _):
            zbuf[i // 8, pl.ds((i % 8) * L, L)] = zvec
            return 0

        lax.fori_loop(0, 128 * 8, zb, 0)
        iota = lax.iota(jnp.int32, L)

        for c in range(pch):
            lo = (s * pch + c) * _CR
            for r in range(zr // 128):
                pltpu.sync_copy(zbuf, acc.at[pl.ds(t * zr + r * 128, 128)])
            plsc.subcore_barrier()

            def scan_blk(arr):
                def blk(i, cnt):
                    eb_off = (t * bpt + i) * _EB
                    pltpu.sync_copy(arr.at[pl.ds(eb_off, _EB)], ebuf)

                    def vec(v, cnt):
                        d = ebuf[pl.ds(v * L, L)]
                        msk = (d >= lo) & (d < lo + _CR)
                        p = eb_off + v * L + iota
                        plsc.store_compressed(pos.at[pl.ds(cnt, L)], p, mask=msk)
                        plsc.store_compressed(off.at[pl.ds(cnt, L)], d - lo,
                                              mask=msk)
                        return cnt + jnp.sum(msk.astype(jnp.int32))

                    return lax.fori_loop(0, _EB // L, vec, cnt)

                return blk

            cnt = lax.fori_loop(0, bpt, scan_blk(c0_hbm), jnp.int32(0))
            cnt = lax.fori_loop(0, bpt, scan_blk(c1_hbm), cnt)

            for j in range(8):
                pos[pl.ds(cnt + j * L, L)] = jnp.zeros((L,), jnp.int32)
                off[pl.ds(cnt + j * L, L)] = jnp.full((L,), _CR, jnp.int32)

            def batch(bi, _):
                base = bi * 128
                for j in range(8):
                    irow[0, pl.ds(j * L, L)] = off[pl.ds(base + j * L, L)]
                pltpu.async_copy(h_hbm.at[pos.at[pl.ds(base, 128)]], rbuf,
                                 sem).wait()
                pltpu.sync_copy(rbuf, acc.at[irow.at[0]], add=True)
                return 0

            lax.fori_loop(0, (cnt + 127) // 128, batch, 0)
            plsc.subcore_barrier()
            for r in range(zr // 128):
                pltpu.sync_copy(acc.at[pl.ds(t * zr + r * 128, 128)],
                                hd_hbm.at[pl.ds(lo + t * zr + r * 128, 128)])
            plsc.subcore_barrier()

    return k


def _scatter_add(h, c0, c1, m):
    n = c0.shape[0]
    n_pad = ((n + NS * _EB - 1) // (NS * _EB)) * (NS * _EB)
    pad = jnp.full((n_pad - n,), 1 << 30, jnp.int32)
    c0p = jnp.concatenate([c0.astype(jnp.int32), pad])
    c1p = jnp.concatenate([c1.astype(jnp.int32), pad])
    return _scatter_call(n_pad, h.shape[0], m)(h, c0p, c1p)


def kernel(feat, child2_0, child2_1, child3_0, child3_1, child4_0, child4_1,
           W_up2, b_up2, W_up3, b_up3, W_up4, b_up4,
           W_down3, b_down3, W_down2, b_down2, W_down1, b_down1):
    n1 = feat.shape[0]
    n2 = child2_0.shape[0]
    n3 = child3_0.shape[0]
    n4 = child4_0.shape[0]

    g0, g1 = _gather2(feat, child2_0, child2_1)
    h2 = _mlp(g0, g1, W_up2, b_up2, n2)
    g0, g1 = _gather2(h2, child3_0, child3_1)
    h3 = _mlp(g0, g1, W_up3, b_up3, n3)
    g0, g1 = _gather2(h3, child4_0, child4_1)
    h4 = _mlp(g0, g1, W_up4, b_up4, n4)

    hd3 = _scatter_add(h4, child4_0, child4_1, n3)
    h3 = _mlp(h3, hd3, W_down3, b_down3, n3)
    hd2 = _scatter_add(h3, child3_0, child3_1, n2)
    h2 = _mlp(h2, hd2, W_down2, b_down2, n2)
    hd1 = _scatter_add(h2, child2_0, child2_1, n1)
    h1 = _mlp(feat, hd1, W_down1, b_down1, n1)
    return h1


# speedup vs baseline: 1.0436x; 1.0436x over previous
import functools

import jax
import jax.numpy as jnp
from jax import lax
from jax.experimental import pallas as pl
from jax.experimental.pallas import tpu as pltpu
from jax.experimental.pallas import tpu_sc as plsc

NC = 2
NS = 16
NW = NC * NS
L = 16
D = 128


_MLP_BLK = 512


@functools.lru_cache(maxsize=None)
def _mlp_call(n_out):
    grid = (pl.cdiv(n_out, _MLP_BLK),)

    def body(a_ref, b_ref, w0_ref, w1_ref, bias_ref, o_ref):
        acc = jnp.dot(a_ref[...], w0_ref[...], preferred_element_type=jnp.float32)
        acc = acc + jnp.dot(b_ref[...], w1_ref[...], preferred_element_type=jnp.float32)
        acc = acc + bias_ref[0:1, :]
        o_ref[...] = acc * jax.nn.sigmoid(acc)

    return pl.pallas_call(
        body,
        grid=grid,
        in_specs=[
            pl.BlockSpec((_MLP_BLK, D), lambda i: (i, 0)),
            pl.BlockSpec((_MLP_BLK, D), lambda i: (i, 0)),
            pl.BlockSpec((D, D), lambda i: (0, 0)),
            pl.BlockSpec((D, D), lambda i: (0, 0)),
            pl.BlockSpec((8, D), lambda i: (0, 0)),
        ],
        out_specs=pl.BlockSpec((_MLP_BLK, D), lambda i: (i, 0)),
        out_shape=jax.ShapeDtypeStruct((n_out, D), jnp.float32),
    )


def _mlp(a, b, W, bias, n_out):
    w0 = W[:D]
    w1 = W[D:]
    bias8 = jnp.broadcast_to(bias[None, :], (8, D))
    return _mlp_call(n_out)(a, b, w0, w1, bias8)


_GB = 256


@functools.lru_cache(maxsize=None)
def _gather_call(n_pad, n_h):
    rows_per_w = n_pad // NW
    n_chunks = rows_per_w // _GB
    mesh = plsc.VectorSubcoreMesh(
        core_axis_name="c", subcore_axis_name="s", num_cores=NC, num_subcores=NS)

    @functools.partial(
        pl.kernel,
        out_type=[jax.ShapeDtypeStruct((n_pad, D), jnp.float32),
                  jax.ShapeDtypeStruct((n_pad, D), jnp.float32)],
        mesh=mesh,
        scratch_types=[
            pltpu.VMEM((_GB,), jnp.int32),
            pltpu.VMEM((_GB, D), jnp.float32),
            pltpu.SemaphoreType.DMA,
        ],
    )
    def k(h_hbm, i0_hbm, i1_hbm, g0_hbm, g1_hbm, ibuf, rbuf, sem):
        w = lax.axis_index("s") * NC + lax.axis_index("c")

        def chunk(ci, _):
            base = w * rows_per_w + ci * _GB
            for ih, gh in ((i0_hbm, g0_hbm), (i1_hbm, g1_hbm)):
                pltpu.sync_copy(ih.at[pl.ds(base, _GB)], ibuf)
                cps = [
                    pltpu.async_copy(
                        h_hbm.at[ibuf.at[pl.ds(j * 128, 128)]],
                        rbuf.at[pl.ds(j * 128, 128)], sem)
                    for j in range(_GB // 128)
                ]
                for cp in cps:
                    cp.wait()
                pltpu.sync_copy(rbuf, gh.at[pl.ds(base, _GB)])
            return 0

        lax.fori_loop(0, n_chunks, chunk, 0)

    return k


def _gather2(h, idx0, idx1):
    n = idx0.shape[0]
    n_pad = ((n + NW * _GB - 1) // (NW * _GB)) * (NW * _GB)
    pad = jnp.zeros((n_pad - n,), jnp.int32)
    i0 = jnp.concatenate([idx0.astype(jnp.int32), pad])
    i1 = jnp.concatenate([idx1.astype(jnp.int32), pad])
    return _gather_call(n_pad, h.shape[0])(h, i0, i1)


_EB = 1280
_CR = 14336
_XLA_SCATTER = True


_DN = lax.GatherDimensionNumbers(
    offset_dims=(), collapsed_slice_dims=(0,), start_index_map=(0,))


def _take16(x, idx):
    return lax.gather(x, idx[:, None], _DN, slice_sizes=(1,),
                      mode=lax.GatherScatterMode.PROMISE_IN_BOUNDS)


@functools.lru_cache(maxsize=None)
def _scatter_call(n_pad, n_h, m):
    pch = pl.cdiv(pl.cdiv(m, NC), _CR)
    m_pad = NC * pch * _CR
    bpt = n_pad // (_EB * NS)
    cap = 2 * bpt * _EB + 128
    mesh = plsc.VectorSubcoreMesh(
        core_axis_name="c", subcore_axis_name="s", num_cores=NC, num_subcores=NS)
    zr = _CR // NS

    @functools.partial(
        pl.kernel,
        out_type=jax.ShapeDtypeStruct((m_pad, D), jnp.float32),
        mesh=mesh,
        scratch_types=[
            pltpu.VMEM((cap,), jnp.int32),
            pltpu.VMEM((cap,), jnp.int32),
            pltpu.VMEM((_EB,), jnp.int32),
            pltpu.VMEM((128, D), jnp.float32),
            pltpu.VMEM((128, D), jnp.float32),
            pltpu.VMEM((1, 128), jnp.int32),
            pltpu.SemaphoreType.DMA,
            pltpu.VMEM_SHARED((_CR + 8, D), jnp.float32),
        ],
    )
    def k(h_hbm, c0_hbm, c1_hbm, z_hbm, hd_hbm, pos, off, ebuf, rbuf, zbuf,
          irow, sem, acc):
        s = lax.axis_index("c")
        t = lax.axis_index("s")
        pltpu.sync_copy(z_hbm, zbuf)
        iota = lax.iota(jnp.int32, L)

        def chunk(c, _c):
            lo = (s * pch + c) * _CR

            def zero(r, _):
                pltpu.sync_copy(zbuf, acc.at[pl.ds(t * zr + r * 128, 128)])
                return 0

            lax.fori_loop(0, zr // 128, zero, 0)
            plsc.subcore_barrier()

            def scan_blk(arr):
                def blk(i, cnt):
                    eb_off = (t * bpt + i) * _EB
                    pltpu.sync_copy(arr.at[pl.ds(eb_off, _EB)], ebuf)

                    def vec(v, cnt):
                        d = ebuf[pl.ds(v * L, L)]
                        msk = (d >= lo) & (d < lo + _CR)
                        pf = msk.astype(jnp.int32)
                        for sh in (1, 2, 4, 8):
                            g = _take16(pf, jnp.maximum(iota - sh, 0))
                            pf = jnp.where(iota >= sh, pf + g, pf)
                        total = pf[L - 1]
                        sel = jnp.zeros((L,), jnp.int32)
                        for sh in (8, 4, 2, 1):
                            probe = sel + sh
                            val = _take16(pf, probe - 1)
                            sel = jnp.where(val < iota + 1, probe, sel)
                        pos[pl.ds(cnt, L)] = eb_off + v * L + sel
                        off[pl.ds(cnt, L)] = _take16(d, sel) - lo
                        return cnt + total

                    return lax.fori_loop(0, _EB // L, vec, cnt)

                return blk

            cnt = lax.fori_loop(0, bpt, scan_blk(c0_hbm), jnp.int32(0))
            cnt = lax.fori_loop(0, bpt, scan_blk(c1_hbm), cnt)

            for j in range(8):
                pos[pl.ds(cnt + j * L, L)] = jnp.zeros((L,), jnp.int32)
                off[pl.ds(cnt + j * L, L)] = jnp.full((L,), _CR, jnp.int32)

            def batch(bi, _):
                base = bi * 128
                for j in range(8):
                    irow[0, pl.ds(j * L, L)] = off[pl.ds(base + j * L, L)]
                pltpu.async_copy(h_hbm.at[pos.at[pl.ds(base, 128)]], rbuf,
                                 sem).wait()
                pltpu.sync_copy(rbuf, acc.at[irow.at[0]], add=True)
                return 0

            lax.fori_loop(0, (cnt + 127) // 128, batch, 0)
            plsc.subcore_barrier()

            def wb(r, _):
                pltpu.sync_copy(acc.at[pl.ds(t * zr + r * 128, 128)],
                                hd_hbm.at[pl.ds(lo + t * zr + r * 128, 128)])
                return 0

            lax.fori_loop(0, zr // 128, wb, 0)
            plsc.subcore_barrier()
            return 0

        lax.fori_loop(0, pch, chunk, 0)

    return k


def _scatter_add(h, c0, c1, m):
    if _XLA_SCATTER:
        return jnp.zeros((m, D), jnp.float32).at[c0].add(h).at[c1].add(h)
    n = c0.shape[0]
    n_pad = ((n + NS * _EB - 1) // (NS * _EB)) * (NS * _EB)
    pad = jnp.full((n_pad - n,), 1 << 30, jnp.int32)
    c0p = jnp.concatenate([c0.astype(jnp.int32), pad])
    c1p = jnp.concatenate([c1.astype(jnp.int32), pad])
    z = jnp.zeros((128, D), jnp.float32)
    return _scatter_call(n_pad, h.shape[0], m)(h, c0p, c1p, z)


def kernel(feat, child2_0, child2_1, child3_0, child3_1, child4_0, child4_1,
           W_up2, b_up2, W_up3, b_up3, W_up4, b_up4,
           W_down3, b_down3, W_down2, b_down2, W_down1, b_down1):
    n1 = feat.shape[0]
    n2 = child2_0.shape[0]
    n3 = child3_0.shape[0]
    n4 = child4_0.shape[0]

    g0, g1 = _gather2(feat, child2_0, child2_1)
    h2 = _mlp(g0, g1, W_up2, b_up2, n2)
    g0, g1 = _gather2(h2, child3_0, child3_1)
    h3 = _mlp(g0, g1, W_up3, b_up3, n3)
    g0, g1 = _gather2(h3, child4_0, child4_1)
    h4 = _mlp(g0, g1, W_up4, b_up4, n4)

    hd3 = _scatter_add(h4, child4_0, child4_1, n3)
    h3 = _mlp(h3, hd3, W_down3, b_down3, n3)
    hd2 = _scatter_add(h3, child3_0, child3_1, n2)
    h2 = _mlp(h2, hd2, W_down2, b_down2, n2)
    hd1 = _scatter_add(h2, child2_0, child2_1, n1)
    h1 = _mlp(feat, hd1, W_down1, b_down1, n1)
    return h1
